# baseline (speedup 1.0000x reference)
import jax
import jax.numpy as jnp
from jax.experimental import pallas as pl
from jax.experimental.pallas import tpu as pltpu

_G = 4
_BM = 256


def _spmm_body(*refs):
    adj_refs = refs[:_G]
    emb_ref = refs[_G]
    out_ref = refs[_G + 1]
    for g in range(_G):
        out_ref[0, g * _BM:(g + 1) * _BM, :] = jnp.dot(
            adj_refs[g][0], emb_ref[...], preferred_element_type=jnp.float32
        )


def kernel(adj, embeds):
    M, K = adj.shape
    _, N = embeds.shape
    nchunk = M // _BM
    steps = nchunk // _G
    adjr = adj.reshape(nchunk, _BM, K)
    in_specs = [
        pl.BlockSpec((1, _BM, K), (lambda i, g=g: (i * _G + g, 0, 0)))
        for g in range(_G)
    ]
    in_specs.append(pl.BlockSpec((K, N), lambda i: (0, 0)))
    out = pl.pallas_call(
        _spmm_body,
        grid=(steps,),
        in_specs=in_specs,
        out_specs=pl.BlockSpec((1, _G * _BM, N), lambda i: (i, 0, 0)),
        out_shape=jax.ShapeDtypeStruct((steps, _G * _BM, N), jnp.float32),
        compiler_params=pltpu.CompilerParams(
            dimension_semantics=("arbitrary",),
        ),
    )(*([adjr] * _G + [embeds]))
    return out.reshape(M, N)
